# baseline (speedup 1.0000x reference)
import functools

import jax
import jax.numpy as jnp
from jax import lax
from jax.experimental import pallas as pl
from jax.experimental.pallas import tpu as pltpu
from jax.experimental.pallas import tpu_sc as plsc

VOCAB = 1000000
D = 64
BATCH = 4096
HIST = 200

NC = 2
NS = 16
NW = NC * NS

B = BATCH * HIST
B_PER_W = B // NW
CH = 128
N_CH = B_PER_W // CH
NBUF = 4


def _make_kernel():
  mesh = plsc.VectorSubcoreMesh(core_axis_name="c", subcore_axis_name="s")

  @functools.partial(
      pl.kernel,
      mesh=mesh,
      out_type=jax.ShapeDtypeStruct((B, D), jnp.float32),
      scratch_types=[
          pltpu.VMEM((N_CH, CH), jnp.int32),
          pltpu.VMEM((NBUF, CH, D), jnp.float32),
      ] + [pltpu.SemaphoreType.DMA] * NBUF,
      compiler_params=pltpu.CompilerParams(use_tc_tiling_on_sc=False),
  )
  def k(idx_hbm, table_hbm, out_hbm, idx_v, rows_v, *gsems):
    wid = lax.axis_index("s") * NC + lax.axis_index("c")
    base = wid * B_PER_W

    pltpu.sync_copy(idx_hbm.at[wid], idx_v)

    def start_gather(chunk, b):
      pltpu.make_async_copy(
          table_hbm.at[idx_v.at[chunk]], rows_v.at[b], gsems[b]
      ).start()

    def wait_gather(b):
      pltpu.make_async_copy(
          table_hbm.at[idx_v.at[0]], rows_v.at[b], gsems[b]
      ).wait()

    for b in range(NBUF):
      start_gather(b, b)

    def outer(t, carry):
      for b in range(NBUF):
        j = t * NBUF + b
        wait_gather(b)
        pltpu.sync_copy(rows_v.at[b], out_hbm.at[pl.ds(base + j * CH, CH)])
        nxt = j + NBUF

        @pl.when(nxt < N_CH)
        def _():
          start_gather(nxt, b)

      return carry

    lax.fori_loop(0, N_CH // NBUF, outer, 0, unroll=False)

  return k


_gather_kernel = _make_kernel()


@jax.jit
def kernel(indices, table):
  idx = indices.reshape(NW, N_CH, CH)
  out = _gather_kernel(idx, table)
  return out.reshape(BATCH, HIST, D)
